# baseline (speedup 1.0000x reference)
import functools
import jax
import jax.numpy as jnp
from jax.experimental import pallas as pl
from jax.experimental.pallas import tpu as pltpu

_EPS = 1e-12


def _rmax_kernel(x_ref, o_ref):
    x = x_ref[...]
    ss = jnp.sum(x * x, axis=-1, keepdims=True)
    xn = x * (1.0 / jnp.maximum(jnp.sqrt(ss), _EPS))
    m = jnp.max(xn, axis=1)
    ss2 = jnp.sum(m * m, axis=-1, keepdims=True)
    o_ref[...] = m * (1.0 / jnp.maximum(jnp.sqrt(ss2), _EPS))


def _mmax_kernel(m1_ref, m2_ref, o1_ref, o2_ref, *, f, d):
    for mr, orf in ((m1_ref, o1_ref), (m2_ref, o2_ref)):
        m = mr[:, 0:d]
        for j in range(1, f):
            m = jnp.maximum(m, mr[:, j * d:(j + 1) * d])
        ss = jnp.sum(m * m, axis=-1, keepdims=True)
        orf[...] = m * (1.0 / jnp.maximum(jnp.sqrt(ss), _EPS))


def _sim_kernel(r_ref, m1_ref, m2_ref, qr_ref, rnd_ref,
                i1_ref, i2_ref, hi_ref, lo_ref,
                bv1, bi1, bv2, bi2, *, mt):
    step = pl.program_id(0)
    r = r_ref[...]
    for m_ref, iref, bv, bi in ((m1_ref, i1_ref, bv1, bi1),
                                (m2_ref, i2_ref, bv2, bi2)):
        mb = m_ref[...]
        sim = jax.lax.dot_general(r, mb, (((1,), (1,)), ((), ())),
                                  preferred_element_type=jnp.float32)
        lmax = jnp.max(sim, axis=1, keepdims=True)
        ii = jax.lax.broadcasted_iota(jnp.int32, sim.shape, 1) + step * mt
        larg = jnp.min(jnp.where(sim == lmax, ii, jnp.int32(2 ** 30)),
                       axis=1, keepdims=True)

        @pl.when(step == 0)
        def _():
            bv[...] = lmax
            bi[...] = larg

        @pl.when(step > 0)
        def _():
            pv = bv[...]
            pi = bi[...]
            take = lmax > pv
            bv[...] = jnp.where(take, lmax, pv)
            bi[...] = jnp.where(take, larg, pi)

        iref[...] = bi[...]

    @pl.when(step == 0)
    def _():
        v = qr_ref[...]
        io = jax.lax.broadcasted_iota(jnp.int32, v.shape, 1)
        mx = jnp.max(v, axis=1, keepdims=True)
        hi_ref[...] = jnp.min(jnp.where(v == mx, io, jnp.int32(2 ** 30)),
                              axis=1, keepdims=True)
        rank = jnp.zeros(v.shape, jnp.int32)
        nf = v.shape[1]
        for j in range(nf):
            cj = v[:, j:j + 1]
            rank = rank + jnp.where((cj < v) | ((cj == v) & (io > j)), 1, 0)
        rsel = rnd_ref[...]
        lo_ref[...] = jnp.sum(jnp.where(rank == rsel, io, 0),
                              axis=1, keepdims=True)


def _gather_kernel(s1, s2, m1_any, m2_any, o1_ref, o2_ref,
                   sc1, sc2, sem1, sem2, *, gb):
    g = pl.program_id(0)
    n = pl.num_programs(0)

    def issue(s, m_any, scr, sem, grp, q):
        for k in range(gb):
            pltpu.make_async_copy(
                m_any.at[pl.ds(s[grp * gb + k], 1), :],
                scr.at[q, pl.ds(k, 1)],
                sem.at[q]).start()

    def wait(s, m_any, scr, sem, grp, q):
        for k in range(gb):
            pltpu.make_async_copy(
                m_any.at[pl.ds(s[grp * gb + k], 1), :],
                scr.at[q, pl.ds(k, 1)],
                sem.at[q]).wait()

    for s, m_any, orf, scr, sem in ((s1, m1_any, o1_ref, sc1, sem1),
                                    (s2, m2_any, o2_ref, sc2, sem2)):
        p = jax.lax.rem(g, 2)

        @pl.when(g == 0)
        def _():
            issue(s, m_any, scr, sem, 0, 0)

        @pl.when(g + 1 < n)
        def _():
            issue(s, m_any, scr, sem, g + 1, jax.lax.rem(g + 1, 2))

        wait(s, m_any, scr, sem, g, p)
        orf[...] = scr[p].reshape(orf.shape)


@jax.jit
def kernel(input_pos_feature, mem1, mem2, QR_frm_weight_p):
    B, T, O, D = input_pos_feature.shape
    M = mem1.shape[0]
    BT = B * T
    F = mem1.shape[1] // D
    x = input_pos_feature.reshape(BT, O, D)

    RB = 16 if BT % 16 == 0 else (8 if BT % 8 == 0 else 1)
    r_maxpool = pl.pallas_call(
        _rmax_kernel,
        grid=(BT // RB,),
        in_specs=[pl.BlockSpec((RB, O, D), lambda i: (i, 0, 0))],
        out_specs=pl.BlockSpec((RB, D), lambda i: (i, 0)),
        out_shape=jax.ShapeDtypeStruct((BT, D), jnp.float32),
        compiler_params=pltpu.CompilerParams(
            dimension_semantics=("arbitrary",)),
    )(x)

    MB = 40 if M % 40 == 0 else (8 if M % 8 == 0 else 1)
    m1n, m2n = pl.pallas_call(
        functools.partial(_mmax_kernel, f=F, d=D),
        grid=(M // MB,),
        in_specs=[pl.BlockSpec((MB, F * D), lambda i: (i, 0)),
                  pl.BlockSpec((MB, F * D), lambda i: (i, 0))],
        out_specs=[pl.BlockSpec((MB, D), lambda i: (i, 0)),
                   pl.BlockSpec((MB, D), lambda i: (i, 0))],
        out_shape=[jax.ShapeDtypeStruct((M, D), jnp.float32),
                   jax.ShapeDtypeStruct((M, D), jnp.float32)],
        compiler_params=pltpu.CompilerParams(
            dimension_semantics=("arbitrary",)),
    )(mem1, mem2)

    nheads = QR_frm_weight_p.shape[1]
    frame_num = QR_frm_weight_p.shape[2]
    BH = B * nheads
    qr = QR_frm_weight_p.reshape(BH, frame_num)
    rand_idx = jax.random.randint(
        jax.random.key(123), (B, nheads, 1), 0, frame_num - 1
    ).reshape(BH, 1)

    MT = 200 if M % 200 == 0 else M
    idx1, idx2, hi, lo = pl.pallas_call(
        functools.partial(_sim_kernel, mt=MT),
        grid=(M // MT,),
        in_specs=[pl.BlockSpec((BT, D), lambda i: (0, 0)),
                  pl.BlockSpec((MT, D), lambda i: (i, 0)),
                  pl.BlockSpec((MT, D), lambda i: (i, 0)),
                  pl.BlockSpec((BH, frame_num), lambda i: (0, 0)),
                  pl.BlockSpec((BH, 1), lambda i: (0, 0))],
        out_specs=[pl.BlockSpec((BT, 1), lambda i: (0, 0)),
                   pl.BlockSpec((BT, 1), lambda i: (0, 0)),
                   pl.BlockSpec((BH, 1), lambda i: (0, 0)),
                   pl.BlockSpec((BH, 1), lambda i: (0, 0))],
        out_shape=[jax.ShapeDtypeStruct((BT, 1), jnp.int32),
                   jax.ShapeDtypeStruct((BT, 1), jnp.int32),
                   jax.ShapeDtypeStruct((BH, 1), jnp.int32),
                   jax.ShapeDtypeStruct((BH, 1), jnp.int32)],
        scratch_shapes=[pltpu.VMEM((BT, 1), jnp.float32),
                        pltpu.VMEM((BT, 1), jnp.int32),
                        pltpu.VMEM((BT, 1), jnp.float32),
                        pltpu.VMEM((BT, 1), jnp.int32)],
        compiler_params=pltpu.CompilerParams(
            dimension_semantics=("arbitrary",)),
    )(r_maxpool, m1n, m2n, qr, rand_idx)

    s1 = idx1.reshape(BT)
    s2 = idx2.reshape(BT)

    GB = 16 if BT % 16 == 0 else (8 if BT % 8 == 0 else 1)
    rp, rn = pl.pallas_call(
        functools.partial(_gather_kernel, gb=GB),
        grid_spec=pltpu.PrefetchScalarGridSpec(
            num_scalar_prefetch=2,
            grid=(BT // GB,),
            in_specs=[pl.BlockSpec(memory_space=pl.ANY),
                      pl.BlockSpec(memory_space=pl.ANY)],
            out_specs=[pl.BlockSpec((GB, F, D), lambda i, a, b: (i, 0, 0)),
                       pl.BlockSpec((GB, F, D), lambda i, a, b: (i, 0, 0))],
            scratch_shapes=[pltpu.VMEM((2, GB, F * D), jnp.float32),
                            pltpu.VMEM((2, GB, F * D), jnp.float32),
                            pltpu.SemaphoreType.DMA((2,)),
                            pltpu.SemaphoreType.DMA((2,))],
        ),
        out_shape=[jax.ShapeDtypeStruct((BT, F, D), jnp.float32),
                   jax.ShapeDtypeStruct((BT, F, D), jnp.float32)],
        compiler_params=pltpu.CompilerParams(
            dimension_semantics=("arbitrary",)),
    )(s1, s2, mem1, mem2)
    Rp_updated = rp.reshape(B, T, O, D)
    Rn_updated = rn.reshape(B, T, O, D)
    high = hi.reshape(B, nheads, 1)
    low = lo.reshape(B, nheads, 1)
    return (Rp_updated, Rn_updated, high, low)


# speedup vs baseline: 1.0104x; 1.0104x over previous
import functools
import jax
import jax.numpy as jnp
from jax.experimental import pallas as pl
from jax.experimental.pallas import tpu as pltpu

_EPS = 1e-12


def _rmax_kernel(x_ref, o_ref):
    x = x_ref[...]
    ss = jnp.sum(x * x, axis=-1, keepdims=True)
    xn = x / jnp.maximum(jnp.sqrt(ss), _EPS)
    m = jnp.max(xn, axis=1)
    ss2 = jnp.sum(m * m, axis=-1, keepdims=True)
    o_ref[...] = m / jnp.maximum(jnp.sqrt(ss2), _EPS)


def _mmax_kernel(m1a_ref, m1b_ref, m2a_ref, m2b_ref, o1_ref, o2_ref, *, f, d):
    fh = f // 2
    for ma, mb_, orf in ((m1a_ref, m1b_ref, o1_ref), (m2a_ref, m2b_ref, o2_ref)):
        m = ma[:, 0:d]
        for j in range(1, fh):
            m = jnp.maximum(m, ma[:, j * d:(j + 1) * d])
        for j in range(f - fh):
            m = jnp.maximum(m, mb_[:, j * d:(j + 1) * d])
        ss = jnp.sum(m * m, axis=-1, keepdims=True)
        orf[...] = m / jnp.maximum(jnp.sqrt(ss), _EPS)


def _sim_kernel(r_ref, m1_ref, m2_ref, qr_ref, rnd_ref,
                i1_ref, i2_ref, hi_ref, lo_ref,
                bv1, bi1, bv2, bi2, *, mt):
    step = pl.program_id(0)
    r = r_ref[...]
    for m_ref, iref, bv, bi in ((m1_ref, i1_ref, bv1, bi1),
                                (m2_ref, i2_ref, bv2, bi2)):
        mb = m_ref[...]
        sim = jax.lax.dot_general(r, mb, (((1,), (1,)), ((), ())),
                                  preferred_element_type=jnp.float32)
        lmax = jnp.max(sim, axis=1, keepdims=True)
        ii = jax.lax.broadcasted_iota(jnp.int32, sim.shape, 1) + step * mt
        larg = jnp.min(jnp.where(sim == lmax, ii, jnp.int32(2 ** 30)),
                       axis=1, keepdims=True)

        @pl.when(step == 0)
        def _():
            bv[...] = lmax
            bi[...] = larg

        @pl.when(step > 0)
        def _():
            pv = bv[...]
            pi = bi[...]
            take = lmax > pv
            bv[...] = jnp.where(take, lmax, pv)
            bi[...] = jnp.where(take, larg, pi)

        iref[...] = bi[...]

    @pl.when(step == 0)
    def _():
        v = qr_ref[...]
        io = jax.lax.broadcasted_iota(jnp.int32, v.shape, 1)
        mx = jnp.max(v, axis=1, keepdims=True)
        hi_ref[...] = jnp.min(jnp.where(v == mx, io, jnp.int32(2 ** 30)),
                              axis=1, keepdims=True)
        rank = jnp.zeros(v.shape, jnp.int32)
        nf = v.shape[1]
        for j in range(nf):
            cj = v[:, j:j + 1]
            rank = rank + jnp.where((cj < v) | ((cj == v) & (io > j)), 1, 0)
        rsel = rnd_ref[...]
        lo_ref[...] = jnp.sum(jnp.where(rank == rsel, io, 0),
                              axis=1, keepdims=True)


def _gather_kernel(s1, s2, m1_any, m2_any, o1_ref, o2_ref,
                   sc1, sc2, sem1, sem2, *, gb):
    g = pl.program_id(0)
    n = pl.num_programs(0)

    def issue(s, m_any, scr, sem, grp, q):
        for k in range(gb):
            pltpu.make_async_copy(
                m_any.at[pl.ds(s[grp * gb + k], 1), :],
                scr.at[q, pl.ds(k, 1)],
                sem.at[q]).start()

    def wait(s, m_any, scr, sem, grp, q):
        for k in range(gb):
            pltpu.make_async_copy(
                m_any.at[pl.ds(s[grp * gb + k], 1), :],
                scr.at[q, pl.ds(k, 1)],
                sem.at[q]).wait()

    for s, m_any, orf, scr, sem in ((s1, m1_any, o1_ref, sc1, sem1),
                                    (s2, m2_any, o2_ref, sc2, sem2)):
        p = jax.lax.rem(g, 2)

        @pl.when(g == 0)
        def _():
            issue(s, m_any, scr, sem, 0, 0)

        @pl.when(g + 1 < n)
        def _():
            issue(s, m_any, scr, sem, g + 1, jax.lax.rem(g + 1, 2))

        wait(s, m_any, scr, sem, g, p)
        orf[...] = scr[p].reshape(orf.shape)


@jax.jit
def kernel(input_pos_feature, mem1, mem2, QR_frm_weight_p):
    B, T, O, D = input_pos_feature.shape
    M = mem1.shape[0]
    BT = B * T
    F = mem1.shape[1] // D
    x = input_pos_feature.reshape(BT, O, D)

    RB = 40 if BT % 40 == 0 else (8 if BT % 8 == 0 else 1)
    r_maxpool = pl.pallas_call(
        _rmax_kernel,
        grid=(BT // RB,),
        in_specs=[pl.BlockSpec((RB, O, D), lambda i: (i, 0, 0))],
        out_specs=pl.BlockSpec((RB, D), lambda i: (i, 0)),
        out_shape=jax.ShapeDtypeStruct((BT, D), jnp.float32),
        compiler_params=pltpu.CompilerParams(
            dimension_semantics=("arbitrary",)),
    )(x)

    MB = 40 if M % 40 == 0 else (8 if M % 8 == 0 else 1)
    HD = (F // 2) * D
    m1n, m2n = pl.pallas_call(
        functools.partial(_mmax_kernel, f=F, d=D),
        grid=(M // MB,),
        in_specs=[pl.BlockSpec((MB, HD), lambda i: (i, 0)),
                  pl.BlockSpec((MB, HD), lambda i: (i, 1)),
                  pl.BlockSpec((MB, HD), lambda i: (i, 0)),
                  pl.BlockSpec((MB, HD), lambda i: (i, 1))],
        out_specs=[pl.BlockSpec((MB, D), lambda i: (i, 0)),
                   pl.BlockSpec((MB, D), lambda i: (i, 0))],
        out_shape=[jax.ShapeDtypeStruct((M, D), jnp.float32),
                   jax.ShapeDtypeStruct((M, D), jnp.float32)],
        compiler_params=pltpu.CompilerParams(
            dimension_semantics=("arbitrary",)),
    )(mem1, mem1, mem2, mem2)

    nheads = QR_frm_weight_p.shape[1]
    frame_num = QR_frm_weight_p.shape[2]
    BH = B * nheads
    qr = QR_frm_weight_p.reshape(BH, frame_num)
    rand_idx = jax.random.randint(
        jax.random.key(123), (B, nheads, 1), 0, frame_num - 1
    ).reshape(BH, 1)

    MT = 200 if M % 200 == 0 else M
    idx1, idx2, hi, lo = pl.pallas_call(
        functools.partial(_sim_kernel, mt=MT),
        grid=(M // MT,),
        in_specs=[pl.BlockSpec((BT, D), lambda i: (0, 0)),
                  pl.BlockSpec((MT, D), lambda i: (i, 0)),
                  pl.BlockSpec((MT, D), lambda i: (i, 0)),
                  pl.BlockSpec((BH, frame_num), lambda i: (0, 0)),
                  pl.BlockSpec((BH, 1), lambda i: (0, 0))],
        out_specs=[pl.BlockSpec((BT, 1), lambda i: (0, 0)),
                   pl.BlockSpec((BT, 1), lambda i: (0, 0)),
                   pl.BlockSpec((BH, 1), lambda i: (0, 0)),
                   pl.BlockSpec((BH, 1), lambda i: (0, 0))],
        out_shape=[jax.ShapeDtypeStruct((BT, 1), jnp.int32),
                   jax.ShapeDtypeStruct((BT, 1), jnp.int32),
                   jax.ShapeDtypeStruct((BH, 1), jnp.int32),
                   jax.ShapeDtypeStruct((BH, 1), jnp.int32)],
        scratch_shapes=[pltpu.VMEM((BT, 1), jnp.float32),
                        pltpu.VMEM((BT, 1), jnp.int32),
                        pltpu.VMEM((BT, 1), jnp.float32),
                        pltpu.VMEM((BT, 1), jnp.int32)],
        compiler_params=pltpu.CompilerParams(
            dimension_semantics=("arbitrary",)),
    )(r_maxpool, m1n, m2n, qr, rand_idx)

    s1 = idx1.reshape(BT)
    s2 = idx2.reshape(BT)

    GB = 16 if BT % 16 == 0 else (8 if BT % 8 == 0 else 1)
    rp, rn = pl.pallas_call(
        functools.partial(_gather_kernel, gb=GB),
        grid_spec=pltpu.PrefetchScalarGridSpec(
            num_scalar_prefetch=2,
            grid=(BT // GB,),
            in_specs=[pl.BlockSpec(memory_space=pl.ANY),
                      pl.BlockSpec(memory_space=pl.ANY)],
            out_specs=[pl.BlockSpec((GB, F, D), lambda i, a, b: (i, 0, 0)),
                       pl.BlockSpec((GB, F, D), lambda i, a, b: (i, 0, 0))],
            scratch_shapes=[pltpu.VMEM((2, GB, F * D), jnp.float32),
                            pltpu.VMEM((2, GB, F * D), jnp.float32),
                            pltpu.SemaphoreType.DMA((2,)),
                            pltpu.SemaphoreType.DMA((2,))],
        ),
        out_shape=[jax.ShapeDtypeStruct((BT, F, D), jnp.float32),
                   jax.ShapeDtypeStruct((BT, F, D), jnp.float32)],
        compiler_params=pltpu.CompilerParams(
            dimension_semantics=("arbitrary",)),
    )(s1, s2, mem1, mem2)
    Rp_updated = rp.reshape(B, T, O, D)
    Rn_updated = rn.reshape(B, T, O, D)
    high = hi.reshape(B, nheads, 1)
    low = lo.reshape(B, nheads, 1)
    return (Rp_updated, Rn_updated, high, low)


# speedup vs baseline: 1.0286x; 1.0180x over previous
import functools
import jax
import jax.numpy as jnp
from jax.experimental import pallas as pl
from jax.experimental.pallas import tpu as pltpu

_EPS = 1e-12


def _rmax_kernel(x_ref, o_ref):
    x = x_ref[...]
    ss = jnp.sum(x * x, axis=-1, keepdims=True)
    xn = x / jnp.maximum(jnp.sqrt(ss), _EPS)
    m = jnp.max(xn, axis=1)
    ss2 = jnp.sum(m * m, axis=-1, keepdims=True)
    o_ref[...] = m / jnp.maximum(jnp.sqrt(ss2), _EPS)


def _mmax_kernel(m1a_ref, m1b_ref, m2a_ref, m2b_ref, o1_ref, o2_ref, *, f, d):
    fh = f // 2
    for ma, mb_, orf in ((m1a_ref, m1b_ref, o1_ref), (m2a_ref, m2b_ref, o2_ref)):
        m = ma[:, 0:d]
        for j in range(1, fh):
            m = jnp.maximum(m, ma[:, j * d:(j + 1) * d])
        for j in range(f - fh):
            m = jnp.maximum(m, mb_[:, j * d:(j + 1) * d])
        ss = jnp.sum(m * m, axis=-1, keepdims=True)
        orf[...] = m / jnp.maximum(jnp.sqrt(ss), _EPS)


def _sim_kernel(r_ref, m1_ref, m2_ref, qr_ref, rnd_ref,
                i1_ref, i2_ref, hi_ref, lo_ref,
                bv1, bi1, bv2, bi2, *, mt):
    step = pl.program_id(0)
    r = r_ref[...]
    for m_ref, iref, bv, bi in ((m1_ref, i1_ref, bv1, bi1),
                                (m2_ref, i2_ref, bv2, bi2)):
        mb = m_ref[...]
        sim = jax.lax.dot_general(r, mb, (((1,), (1,)), ((), ())),
                                  preferred_element_type=jnp.float32)
        lmax = jnp.max(sim, axis=1, keepdims=True)
        ii = jax.lax.broadcasted_iota(jnp.int32, sim.shape, 1) + step * mt
        larg = jnp.min(jnp.where(sim == lmax, ii, jnp.int32(2 ** 30)),
                       axis=1, keepdims=True)

        @pl.when(step == 0)
        def _():
            bv[...] = lmax
            bi[...] = larg

        @pl.when(step > 0)
        def _():
            pv = bv[...]
            pi = bi[...]
            take = lmax > pv
            bv[...] = jnp.where(take, lmax, pv)
            bi[...] = jnp.where(take, larg, pi)

        iref[...] = bi[...]

    @pl.when(step == 0)
    def _():
        v = qr_ref[...]
        io = jax.lax.broadcasted_iota(jnp.int32, v.shape, 1)
        mx = jnp.max(v, axis=1, keepdims=True)
        hi_ref[...] = jnp.min(jnp.where(v == mx, io, jnp.int32(2 ** 30)),
                              axis=1, keepdims=True)
        rank = jnp.zeros(v.shape, jnp.int32)
        nf = v.shape[1]
        for j in range(nf):
            cj = v[:, j:j + 1]
            rank = rank + jnp.where((cj < v) | ((cj == v) & (io > j)), 1, 0)
        rsel = rnd_ref[...]
        lo_ref[...] = jnp.sum(jnp.where(rank == rsel, io, 0),
                              axis=1, keepdims=True)


def _gather_kernel(s1, s2, m1_any, m2_any, o1_ref, o2_ref,
                   sc1, sc2, sem1, sem2, *, gb):
    g = pl.program_id(0)
    n = pl.num_programs(0)

    def issue(s, m_any, scr, sem, grp, q):
        for k in range(gb):
            pltpu.make_async_copy(
                m_any.at[pl.ds(s[grp * gb + k], 1), :],
                scr.at[q, pl.ds(k, 1)],
                sem.at[q]).start()

    def wait(s, m_any, scr, sem, grp, q):
        for k in range(gb):
            pltpu.make_async_copy(
                m_any.at[pl.ds(s[grp * gb + k], 1), :],
                scr.at[q, pl.ds(k, 1)],
                sem.at[q]).wait()

    for s, m_any, orf, scr, sem in ((s1, m1_any, o1_ref, sc1, sem1),
                                    (s2, m2_any, o2_ref, sc2, sem2)):
        p = jax.lax.rem(g, 2)

        @pl.when(g == 0)
        def _():
            issue(s, m_any, scr, sem, 0, 0)

        @pl.when(g + 1 < n)
        def _():
            issue(s, m_any, scr, sem, g + 1, jax.lax.rem(g + 1, 2))

        wait(s, m_any, scr, sem, g, p)
        orf[...] = scr[p].reshape(orf.shape)


@jax.jit
def kernel(input_pos_feature, mem1, mem2, QR_frm_weight_p):
    B, T, O, D = input_pos_feature.shape
    M = mem1.shape[0]
    BT = B * T
    F = mem1.shape[1] // D
    x = input_pos_feature.reshape(BT, O, D)

    RB = 40 if BT % 40 == 0 else (8 if BT % 8 == 0 else 1)
    r_maxpool = pl.pallas_call(
        _rmax_kernel,
        grid=(BT // RB,),
        in_specs=[pl.BlockSpec((RB, O, D), lambda i: (i, 0, 0))],
        out_specs=pl.BlockSpec((RB, D), lambda i: (i, 0)),
        out_shape=jax.ShapeDtypeStruct((BT, D), jnp.float32),
        compiler_params=pltpu.CompilerParams(
            dimension_semantics=("arbitrary",)),
    )(x)

    MB = 40 if M % 40 == 0 else (8 if M % 8 == 0 else 1)
    HD = (F // 2) * D
    m1n, m2n = pl.pallas_call(
        functools.partial(_mmax_kernel, f=F, d=D),
        grid=(M // MB,),
        in_specs=[pl.BlockSpec((MB, HD), lambda i: (i, 0)),
                  pl.BlockSpec((MB, HD), lambda i: (i, 1)),
                  pl.BlockSpec((MB, HD), lambda i: (i, 0)),
                  pl.BlockSpec((MB, HD), lambda i: (i, 1))],
        out_specs=[pl.BlockSpec((MB, D), lambda i: (i, 0)),
                   pl.BlockSpec((MB, D), lambda i: (i, 0))],
        out_shape=[jax.ShapeDtypeStruct((M, D), jnp.float32),
                   jax.ShapeDtypeStruct((M, D), jnp.float32)],
        compiler_params=pltpu.CompilerParams(
            dimension_semantics=("arbitrary",)),
    )(mem1, mem1, mem2, mem2)

    nheads = QR_frm_weight_p.shape[1]
    frame_num = QR_frm_weight_p.shape[2]
    BH = B * nheads
    qr = QR_frm_weight_p.reshape(BH, frame_num)
    rand_idx = jax.random.randint(
        jax.random.key(123), (B, nheads, 1), 0, frame_num - 1
    ).reshape(BH, 1)

    MT = M
    idx1, idx2, hi, lo = pl.pallas_call(
        functools.partial(_sim_kernel, mt=MT),
        grid=(M // MT,),
        in_specs=[pl.BlockSpec((BT, D), lambda i: (0, 0)),
                  pl.BlockSpec((MT, D), lambda i: (i, 0)),
                  pl.BlockSpec((MT, D), lambda i: (i, 0)),
                  pl.BlockSpec((BH, frame_num), lambda i: (0, 0)),
                  pl.BlockSpec((BH, 1), lambda i: (0, 0))],
        out_specs=[pl.BlockSpec((BT, 1), lambda i: (0, 0)),
                   pl.BlockSpec((BT, 1), lambda i: (0, 0)),
                   pl.BlockSpec((BH, 1), lambda i: (0, 0)),
                   pl.BlockSpec((BH, 1), lambda i: (0, 0))],
        out_shape=[jax.ShapeDtypeStruct((BT, 1), jnp.int32),
                   jax.ShapeDtypeStruct((BT, 1), jnp.int32),
                   jax.ShapeDtypeStruct((BH, 1), jnp.int32),
                   jax.ShapeDtypeStruct((BH, 1), jnp.int32)],
        scratch_shapes=[pltpu.VMEM((BT, 1), jnp.float32),
                        pltpu.VMEM((BT, 1), jnp.int32),
                        pltpu.VMEM((BT, 1), jnp.float32),
                        pltpu.VMEM((BT, 1), jnp.int32)],
        compiler_params=pltpu.CompilerParams(
            dimension_semantics=("arbitrary",)),
    )(r_maxpool, m1n, m2n, qr, rand_idx)

    s1 = idx1.reshape(BT)
    s2 = idx2.reshape(BT)

    GB = 16 if BT % 16 == 0 else (8 if BT % 8 == 0 else 1)
    rp, rn = pl.pallas_call(
        functools.partial(_gather_kernel, gb=GB),
        grid_spec=pltpu.PrefetchScalarGridSpec(
            num_scalar_prefetch=2,
            grid=(BT // GB,),
            in_specs=[pl.BlockSpec(memory_space=pl.ANY),
                      pl.BlockSpec(memory_space=pl.ANY)],
            out_specs=[pl.BlockSpec((GB, F, D), lambda i, a, b: (i, 0, 0)),
                       pl.BlockSpec((GB, F, D), lambda i, a, b: (i, 0, 0))],
            scratch_shapes=[pltpu.VMEM((2, GB, F * D), jnp.float32),
                            pltpu.VMEM((2, GB, F * D), jnp.float32),
                            pltpu.SemaphoreType.DMA((2,)),
                            pltpu.SemaphoreType.DMA((2,))],
        ),
        out_shape=[jax.ShapeDtypeStruct((BT, F, D), jnp.float32),
                   jax.ShapeDtypeStruct((BT, F, D), jnp.float32)],
        compiler_params=pltpu.CompilerParams(
            dimension_semantics=("arbitrary",)),
    )(s1, s2, mem1, mem2)
    Rp_updated = rp.reshape(B, T, O, D)
    Rn_updated = rn.reshape(B, T, O, D)
    high = hi.reshape(B, nheads, 1)
    low = lo.reshape(B, nheads, 1)
    return (Rp_updated, Rn_updated, high, low)
